# baseline (speedup 1.0000x reference)
import functools
import jax
import jax.numpy as jnp
from jax import lax
from jax.experimental import pallas as pl
from jax.experimental.pallas import tpu as pltpu
from jax.experimental.pallas import tpu_sc as plsc

_N = 16384
_C = 1000

_NW = 32
_TCOLS = 1
_SPW = 128 * _TCOLS
_SC_SAMP = _NW * _SPW
_SC_BASE = _N - _SC_SAMP
_ROWS = 200
_NCH = _C // _ROWS


@functools.partial(
    pl.kernel,
    out_type=jax.ShapeDtypeStruct((_SC_SAMP,), jnp.float32),
    mesh=plsc.VectorSubcoreMesh(core_axis_name="c", subcore_axis_name="s"),
    scratch_types=[
        pltpu.VMEM((_ROWS, 128), jnp.int32),
        pltpu.VMEM((_ROWS, 128), jnp.int32),
        pltpu.VMEM((_ROWS, 128), jnp.float32),
        pltpu.VMEM((_ROWS, 128), jnp.float32),
        pltpu.VMEM((_SPW,), jnp.float32),
        pltpu.SemaphoreType.DMA,
        pltpu.SemaphoreType.DMA,
        pltpu.SemaphoreType.DMA,
        pltpu.SemaphoreType.DMA,
    ],
    compiler_params=pltpu.CompilerParams(use_tc_tiling_on_sc=True),
)
def _sc_products(yt_hbm, yp_hbm, out_hbm, yt0, yt1, yp0, yp1, prod, s0t, s1t, s0p, s1p):
    wid = lax.axis_index("s") * 2 + lax.axis_index("c")
    col0 = _SC_BASE + wid * _SPW

    yt_slots = (yt0, yt1)
    yp_slots = (yp0, yp1)
    t_sems = (s0t, s1t)
    p_sems = (s0p, s1p)

    def copies(col, ch, slot):
        rs = pl.ds(ch * _ROWS, _ROWS)
        cs = pl.ds(col0 + col * 128, 128)
        return (
            pltpu.make_async_copy(yt_hbm.at[rs, cs], yt_slots[slot], t_sems[slot]),
            pltpu.make_async_copy(yp_hbm.at[rs, cs], yp_slots[slot], p_sems[slot]),
        )

    def chunk_acc(slot, carry):
        ytb = yt_slots[slot]
        ypb = yp_slots[slot]

        def row_body(r, c):
            accn, accp = c
            nn, np_ = [], []
            for v in range(8):
                sl = pl.ds(16 * v, 16)
                m = ytb[r, sl] == 1
                t = jnp.exp(ypb[r, sl])
                rr = 1.0 / t
                nn.append(accn[v] + jnp.where(m, 0.0, t))
                np_.append(accp[v] + jnp.where(m, rr, 0.0))
            return tuple(nn), tuple(np_)

        return lax.fori_loop(0, _ROWS, row_body, carry)

    for col in range(_TCOLS):
        for ch in range(min(2, _NCH)):
            for c in copies(col, ch, ch):
                c.start()
        zero = jnp.zeros((16,), jnp.float32)
        carry = (tuple(zero for _ in range(8)), tuple(zero for _ in range(8)))
        for ch in range(_NCH):
            slot = ch % 2
            for c in copies(col, ch, slot):
                c.wait()
            if ch + 2 < _NCH:
                for c in copies(col, ch + 2, slot):
                    c.start()
            carry = chunk_acc(slot, carry)
        accn, accp = carry
        for v in range(8):
            prod[pl.ds(col * 128 + 16 * v, 16)] = accn[v] * accp[v]

    pltpu.sync_copy(prod, out_hbm.at[pl.ds(wid * _SPW, _SPW)])


_CW = 256
_NBUF = 16
_NCHUNK = _SC_BASE // _CW


def _chunk_sum(yt, yp):
    is_pos = yt == 1
    t = jnp.exp(yp)
    r = 1.0 / t
    s_neg = jnp.sum(jnp.where(is_pos, 0.0, t), axis=0)
    s_pos = jnp.sum(jnp.where(is_pos, r, 0.0), axis=0)
    return jnp.sum(jnp.log1p(s_neg * s_pos))


def _tc_body(yt_hbm, yp_hbm, out_ref, yt_buf, yp_buf, yt_sem, yp_sem):
    def start(chunk, slot):
        pltpu.make_async_copy(
            yt_hbm.at[:, pl.ds(chunk * _CW, _CW)], yt_buf.at[slot], yt_sem.at[slot]
        ).start()
        pltpu.make_async_copy(
            yp_hbm.at[:, pl.ds(chunk * _CW, _CW)], yp_buf.at[slot], yp_sem.at[slot]
        ).start()

    for i in range(_NBUF):
        start(i, i)

    def step(i, acc):
        slot = lax.rem(i, _NBUF)
        pltpu.make_async_copy(
            yt_hbm.at[:, pl.ds(0, _CW)], yt_buf.at[slot], yt_sem.at[slot]
        ).wait()
        pltpu.make_async_copy(
            yp_hbm.at[:, pl.ds(0, _CW)], yp_buf.at[slot], yp_sem.at[slot]
        ).wait()
        cs = _chunk_sum(yt_buf[slot], yp_buf[slot])

        @pl.when(i + _NBUF < _NCHUNK)
        def _():
            start(i + _NBUF, slot)

        return acc + cs

    acc = lax.fori_loop(0, _NCHUNK, step, jnp.float32(0.0))
    out_ref[0, 0] = acc


def _tc_partial(y_true_t, y_pred_t):
    return pl.pallas_call(
        _tc_body,
        in_specs=[
            pl.BlockSpec(memory_space=pl.ANY),
            pl.BlockSpec(memory_space=pl.ANY),
        ],
        out_specs=pl.BlockSpec(memory_space=pltpu.SMEM),
        out_shape=jax.ShapeDtypeStruct((1, 1), jnp.float32),
        scratch_shapes=[
            pltpu.VMEM((_NBUF, _C, _CW), jnp.int32),
            pltpu.VMEM((_NBUF, _C, _CW), jnp.float32),
            pltpu.SemaphoreType.DMA((_NBUF,)),
            pltpu.SemaphoreType.DMA((_NBUF,)),
        ],
        compiler_params=pltpu.CompilerParams(skip_device_barrier=True),
    )(y_true_t, y_pred_t)


def _final_body(part_ref, p_ref, out_ref):
    out_ref[0, 0] = (part_ref[0, 0] + jnp.sum(jnp.log1p(p_ref[...]))) / _N


def kernel(y_true, y_pred):
    yt_t = y_true.T
    yp_t = y_pred.T
    prods = _sc_products(yt_t, yp_t)
    partial = _tc_partial(yt_t, yp_t)
    out = pl.pallas_call(
        _final_body,
        in_specs=[
            pl.BlockSpec(memory_space=pltpu.SMEM),
            pl.BlockSpec(memory_space=pltpu.VMEM),
        ],
        out_specs=pl.BlockSpec(memory_space=pltpu.SMEM),
        out_shape=jax.ShapeDtypeStruct((1, 1), jnp.float32),
    )(partial, prods.reshape(32, 128))
    return out[0, 0].reshape(1)


# speedup vs baseline: 1.4069x; 1.4069x over previous
import jax
import jax.numpy as jnp
from jax import lax
from jax.experimental import pallas as pl
from jax.experimental.pallas import tpu as pltpu

_N = 16384
_C = 1000
_CW = 256
_NBUF = 16
_NCHUNK = _N // _CW


def _chunk_sum(yt, yp):
    is_pos = yt == 1
    t = jnp.exp(yp)
    r = 1.0 / t
    s_neg = jnp.sum(jnp.where(is_pos, 0.0, t), axis=0)
    s_pos = jnp.sum(jnp.where(is_pos, r, 0.0), axis=0)
    return jnp.sum(jnp.log1p(s_neg * s_pos))


def _body(yt_hbm, yp_hbm, out_ref, yt_buf, yp_buf, yt_sem, yp_sem):
    def start(chunk, slot):
        pltpu.make_async_copy(
            yt_hbm.at[:, pl.ds(chunk * _CW, _CW)], yt_buf.at[slot], yt_sem.at[slot]
        ).start()
        pltpu.make_async_copy(
            yp_hbm.at[:, pl.ds(chunk * _CW, _CW)], yp_buf.at[slot], yp_sem.at[slot]
        ).start()

    for i in range(_NBUF):
        start(i, i)

    def step(i, acc):
        slot = lax.rem(i, _NBUF)
        pltpu.make_async_copy(
            yt_hbm.at[:, pl.ds(0, _CW)], yt_buf.at[slot], yt_sem.at[slot]
        ).wait()
        pltpu.make_async_copy(
            yp_hbm.at[:, pl.ds(0, _CW)], yp_buf.at[slot], yp_sem.at[slot]
        ).wait()
        cs = _chunk_sum(yt_buf[slot], yp_buf[slot])

        @pl.when(i + _NBUF < _NCHUNK)
        def _():
            start(i + _NBUF, slot)

        return acc + cs

    acc = lax.fori_loop(0, _NCHUNK, step, jnp.float32(0.0))
    out_ref[0, 0] = acc / _N


def kernel(y_true, y_pred):
    out = pl.pallas_call(
        _body,
        in_specs=[
            pl.BlockSpec(memory_space=pl.ANY),
            pl.BlockSpec(memory_space=pl.ANY),
        ],
        out_specs=pl.BlockSpec(memory_space=pltpu.SMEM),
        out_shape=jax.ShapeDtypeStruct((1, 1), jnp.float32),
        scratch_shapes=[
            pltpu.VMEM((_NBUF, _C, _CW), jnp.int32),
            pltpu.VMEM((_NBUF, _C, _CW), jnp.float32),
            pltpu.SemaphoreType.DMA((_NBUF,)),
            pltpu.SemaphoreType.DMA((_NBUF,)),
        ],
    )(y_true.T, y_pred.T)
    return out[0, 0].reshape(1)
